# baseline (speedup 1.0000x reference)
import functools

import jax
import jax.numpy as jnp
from jax import lax
from jax.experimental import pallas as pl
from jax.experimental.pallas import tpu as pltpu
from jax.experimental.pallas import tpu_sc as plsc

D_MODEL = 64
LANES = 16
NUM_WORKERS = 32
GATHER_CHUNK = 100


def _positional_encoding(max_len, d_model):
    depth = d_model // 2
    positions = jnp.arange(max_len, dtype=jnp.float32)[:, None]
    depths = jnp.arange(depth, dtype=jnp.float32)[None, :] / depth
    angle_rates = 1.0 / (10000.0 ** depths)
    angle_rads = positions * angle_rates
    return jnp.concatenate(
        [jnp.sin(angle_rads), jnp.cos(angle_rads)], axis=-1
    ).astype(jnp.float32)


def _embed_body(seq_ref, table_ref, pos_ref, out_ref, idx_v, pos_v, buf, gsem):
    nc = 2
    wid = lax.axis_index("s") * nc + lax.axis_index("c")
    seqs_per_w = idx_v.shape[0] // 2
    seq_len = 2 * GATHER_CHUNK
    groups = D_MODEL // LANES
    scale = jnp.float32(8.0)

    pltpu.sync_copy(seq_ref.at[wid], idx_v)
    pltpu.sync_copy(pos_ref, pos_v)

    base_row = wid * (seqs_per_w * seq_len)

    @pl.loop(0, seqs_per_w)
    def _seq_loop(s):
        cp0 = pltpu.async_copy(
            table_ref.at[idx_v.at[2 * s]], buf.at[pl.ds(0, GATHER_CHUNK)], gsem
        )
        cp1 = pltpu.async_copy(
            table_ref.at[idx_v.at[2 * s + 1]],
            buf.at[pl.ds(GATHER_CHUNK, GATHER_CHUNK)],
            gsem,
        )
        cp0.wait()
        cp1.wait()

        @pl.loop(0, seq_len)
        def _row_loop(r):
            for g in range(groups):
                sl = pl.ds(g * LANES, LANES)
                buf[r, sl] = buf[r, sl] * scale + pos_v[r, sl]

        pltpu.sync_copy(buf, out_ref.at[pl.ds(base_row + s * seq_len, seq_len)])


def kernel(sequences, embedding_table):
    batch, seq_len = sequences.shape
    vocab, d_model = embedding_table.shape
    assert d_model == D_MODEL and seq_len == 2 * GATHER_CHUNK
    total = batch * seq_len
    per_w = total // NUM_WORKERS
    assert per_w % seq_len == 0

    pos = _positional_encoding(seq_len, d_model)
    seq3 = sequences.reshape(NUM_WORKERS, 2 * (per_w // seq_len), GATHER_CHUNK)
    seq3 = seq3.astype(jnp.int32)

    mesh = plsc.VectorSubcoreMesh(core_axis_name="c", subcore_axis_name="s")
    out = pl.kernel(
        _embed_body,
        out_type=jax.ShapeDtypeStruct((total, d_model), jnp.float32),
        mesh=mesh,
        compiler_params=pltpu.CompilerParams(use_tc_tiling_on_sc=False),
        scratch_types=[
            pltpu.VMEM((2 * (per_w // seq_len), GATHER_CHUNK), jnp.int32),
            pltpu.VMEM((seq_len, d_model), jnp.float32),
            pltpu.VMEM((seq_len, d_model), jnp.float32),
            pltpu.SemaphoreType.DMA,
        ],
    )(seq3, embedding_table, pos)
    return out.reshape(batch, seq_len, d_model)
